# baseline (speedup 1.0000x reference)
import jax
import jax.numpy as jnp
from jax import lax
from jax.experimental import pallas as pl
from jax.experimental.pallas import tpu as pltpu

N = 1500
D = 2
HID = 32


def _fused(feats_ref, adj_ref, cand_ref, mask_ref, act_ref,
           w00, b00, w01, b01, w02, b02,
           w10, b10, w11, b11, w12, b12,
           aw0, ab0, aw1, ab1, aw2, ab2,
           cw0, cb0, cw1, cb1, pm,
           out_ref):
    f32 = jnp.float32
    adj = adj_ref[0]
    feats = feats_ref[0]

    pooled = jnp.dot(adj, feats, preferred_element_type=f32, precision=jax.lax.Precision.HIGHEST) + feats
    t = jnp.maximum(jnp.dot(pooled, w00[...], preferred_element_type=f32, precision=jax.lax.Precision.HIGHEST) + b00[...], 0.0)
    t = jnp.maximum(jnp.dot(t, w01[...], preferred_element_type=f32, precision=jax.lax.Precision.HIGHEST) + b01[...], 0.0)
    t = jnp.dot(t, w02[...], preferred_element_type=f32, precision=jax.lax.Precision.HIGHEST) + b02[...]
    h = jnp.maximum(t, 0.0)

    pooled = jnp.dot(adj, h, preferred_element_type=f32, precision=jax.lax.Precision.HIGHEST) + h
    t = jnp.maximum(jnp.dot(pooled, w10[...], preferred_element_type=f32, precision=jax.lax.Precision.HIGHEST) + b10[...], 0.0)
    t = jnp.maximum(jnp.dot(t, w11[...], preferred_element_type=f32, precision=jax.lax.Precision.HIGHEST) + b11[...], 0.0)
    t = jnp.dot(t, w12[...], preferred_element_type=f32, precision=jax.lax.Precision.HIGHEST) + b12[...]
    h = jnp.maximum(t, 0.0)

    g = jnp.sum(h, axis=0, keepdims=True) * (1.0 / N)

    cand = cand_ref[0]
    cols = lax.broadcasted_iota(jnp.int32, (N, N), 1)
    onehot = (cols == cand).astype(f32)
    job = jnp.dot(onehot, h, preferred_element_type=f32, precision=jax.lax.Precision.HIGHEST)

    cat = jnp.concatenate(
        [job,
         jnp.broadcast_to(g, (N, HID)),
         jnp.broadcast_to(pm[...], (N, HID))], axis=1)

    a = jnp.tanh(jnp.dot(cat, aw0[...], preferred_element_type=f32, precision=jax.lax.Precision.HIGHEST) + ab0[...])
    a = jnp.tanh(jnp.dot(a, aw1[...], preferred_element_type=f32, precision=jax.lax.Precision.HIGHEST) + ab1[...])
    s = jnp.dot(a, aw2[...], preferred_element_type=f32, precision=jax.lax.Precision.HIGHEST) + ab2[...]
    scores = s * 10.0
    mask = mask_ref[0]
    scores = jnp.where(mask != 0.0, -jnp.inf, scores)

    m = jnp.max(scores, axis=0, keepdims=True)
    e = jnp.exp(scores - m)
    logits = e / jnp.sum(e, axis=0, keepdims=True)

    m2 = jnp.max(logits, axis=0, keepdims=True)
    ls2 = m2 + jnp.log(jnp.sum(jnp.exp(logits - m2), axis=0, keepdims=True))
    logp_all = logits - ls2
    p = jnp.exp(logp_all)
    ent = -jnp.sum(p * logp_all, axis=0, keepdims=True)

    ai = act_ref[0, 0, 0]
    rows = lax.broadcasted_iota(jnp.int32, (N, 1), 0)
    logp = jnp.sum(jnp.where(rows == ai, logp_all, 0.0), axis=0, keepdims=True)

    c = jnp.tanh(jnp.dot(g, cw0[...], preferred_element_type=f32, precision=jax.lax.Precision.HIGHEST) + cb0[...])
    v = jnp.dot(c, cw1[...], preferred_element_type=f32, precision=jax.lax.Precision.HIGHEST) + cb1[...]

    lanes = lax.broadcasted_iota(jnp.int32, (1, 1, 128), 2)
    out = jnp.where(lanes == 0, logp[0, 0],
          jnp.where(lanes == 1, ent[0, 0],
          jnp.where(lanes == 2, v[0, 0], 0.0)))
    out_ref[...] = out


def kernel(x, action, enc_W0_0, enc_b0_0, enc_W0_1, enc_b0_1, enc_W0_2, enc_b0_2,
           enc_W1_0, enc_b1_0, enc_W1_1, enc_b1_1, enc_W1_2, enc_b1_2,
           actor_W0, actor_b0, actor_W1, actor_b1, actor_W2, actor_b2,
           critic_W0, critic_b0, critic_W1, critic_b1, pooled_machine):
    B = x.shape[0]
    f32 = jnp.float32
    off = 2
    feats = x[:, off:off + N * D].reshape(B, N, D)
    off += N * D
    adj = x[:, off:off + N * N].reshape(B, N, N)
    off += N * N
    cand = x[:, off:off + N].astype(jnp.int32).reshape(B, N, 1)
    off += N
    mask = x[:, off:off + N].reshape(B, N, 1)
    act3 = action.astype(jnp.int32).reshape(B, 1, 1)

    def row2(v):
        return v.reshape(1, -1).astype(f32)

    per_sample = lambda bs: pl.BlockSpec(bs, lambda b: (b,) + (0,) * (len(bs) - 1))
    shared = lambda arr: pl.BlockSpec(arr.shape, lambda b: (0,) * arr.ndim)

    weights = [enc_W0_0, row2(enc_b0_0), enc_W0_1, row2(enc_b0_1), enc_W0_2, row2(enc_b0_2),
               enc_W1_0, row2(enc_b1_0), enc_W1_1, row2(enc_b1_1), enc_W1_2, row2(enc_b1_2),
               actor_W0, row2(actor_b0), actor_W1, row2(actor_b1), actor_W2, row2(actor_b2),
               critic_W0, row2(critic_b0), critic_W1, row2(critic_b1), row2(pooled_machine)]

    in_specs = [per_sample((1, N, D)), per_sample((1, N, N)),
                per_sample((1, N, 1)), per_sample((1, N, 1)),
                per_sample((1, 1, 1))] + [shared(w) for w in weights]

    out = pl.pallas_call(
        _fused,
        grid=(B,),
        in_specs=in_specs,
        out_specs=pl.BlockSpec((1, 1, 128), lambda b: (b, 0, 0)),
        out_shape=jax.ShapeDtypeStruct((B, 1, 128), f32),
        compiler_params=pltpu.CompilerParams(
            vmem_limit_bytes=120 * 1024 * 1024),
    )(feats, adj, cand, mask, act3, *weights)

    return action, out[:, 0, 0], out[:, 0, 1], out[:, 0, 2:3]


# speedup vs baseline: 1.1630x; 1.1630x over previous
import jax
import jax.numpy as jnp
from jax import lax
from jax.experimental import pallas as pl
from jax.experimental.pallas import tpu as pltpu

N = 1500
D = 2
HID = 32


def _fused(feats_ref, adj_ref, cand_ref, mask_ref, act_ref,
           w00, b00, w01, b01, w02, b02,
           w10, b10, w11, b11, w12, b12,
           aw0, ab0, aw1, ab1, aw2, ab2,
           cw0, cb0, cw1, cb1, pm,
           out_ref):
    f32 = jnp.float32
    bf16 = jnp.bfloat16
    adj = adj_ref[0].astype(bf16)
    feats = feats_ref[0]

    def split_dot(a_b, m):
        m_hi = m.astype(bf16)
        m_lo = (m - m_hi.astype(f32)).astype(bf16)
        return (jnp.dot(a_b, m_hi, preferred_element_type=f32)
                + jnp.dot(a_b, m_lo, preferred_element_type=f32))

    pooled = split_dot(adj, feats) + feats
    t = jnp.maximum(jnp.dot(pooled, w00[...], preferred_element_type=f32, precision=jax.lax.Precision.HIGHEST) + b00[...], 0.0)
    t = jnp.maximum(jnp.dot(t, w01[...], preferred_element_type=f32, precision=jax.lax.Precision.HIGHEST) + b01[...], 0.0)
    t = jnp.dot(t, w02[...], preferred_element_type=f32, precision=jax.lax.Precision.HIGHEST) + b02[...]
    h = jnp.maximum(t, 0.0)

    pooled = split_dot(adj, h) + h
    t = jnp.maximum(jnp.dot(pooled, w10[...], preferred_element_type=f32, precision=jax.lax.Precision.HIGHEST) + b10[...], 0.0)
    t = jnp.maximum(jnp.dot(t, w11[...], preferred_element_type=f32, precision=jax.lax.Precision.HIGHEST) + b11[...], 0.0)
    t = jnp.dot(t, w12[...], preferred_element_type=f32, precision=jax.lax.Precision.HIGHEST) + b12[...]
    h = jnp.maximum(t, 0.0)

    g = jnp.sum(h, axis=0, keepdims=True) * (1.0 / N)

    cand = cand_ref[0]
    cols = lax.broadcasted_iota(jnp.int32, (N, N), 1)
    onehot = (cols == cand).astype(bf16)
    job = split_dot(onehot, h)

    cat = jnp.concatenate(
        [job,
         jnp.broadcast_to(g, (N, HID)),
         jnp.broadcast_to(pm[...], (N, HID))], axis=1)

    a = jnp.tanh(jnp.dot(cat, aw0[...], preferred_element_type=f32, precision=jax.lax.Precision.HIGHEST) + ab0[...])
    a = jnp.tanh(jnp.dot(a, aw1[...], preferred_element_type=f32, precision=jax.lax.Precision.HIGHEST) + ab1[...])
    s = jnp.dot(a, aw2[...], preferred_element_type=f32, precision=jax.lax.Precision.HIGHEST) + ab2[...]
    scores = s * 10.0
    mask = mask_ref[0]
    scores = jnp.where(mask != 0.0, -jnp.inf, scores)

    m = jnp.max(scores, axis=0, keepdims=True)
    e = jnp.exp(scores - m)
    logits = e / jnp.sum(e, axis=0, keepdims=True)

    m2 = jnp.max(logits, axis=0, keepdims=True)
    ls2 = m2 + jnp.log(jnp.sum(jnp.exp(logits - m2), axis=0, keepdims=True))
    logp_all = logits - ls2
    p = jnp.exp(logp_all)
    ent = -jnp.sum(p * logp_all, axis=0, keepdims=True)

    ai = act_ref[0, 0, 0]
    rows = lax.broadcasted_iota(jnp.int32, (N, 1), 0)
    logp = jnp.sum(jnp.where(rows == ai, logp_all, 0.0), axis=0, keepdims=True)

    c = jnp.tanh(jnp.dot(g, cw0[...], preferred_element_type=f32, precision=jax.lax.Precision.HIGHEST) + cb0[...])
    v = jnp.dot(c, cw1[...], preferred_element_type=f32, precision=jax.lax.Precision.HIGHEST) + cb1[...]

    lanes = lax.broadcasted_iota(jnp.int32, (1, 1, 128), 2)
    out = jnp.where(lanes == 0, logp[0, 0],
          jnp.where(lanes == 1, ent[0, 0],
          jnp.where(lanes == 2, v[0, 0], 0.0)))
    out_ref[...] = out


def kernel(x, action, enc_W0_0, enc_b0_0, enc_W0_1, enc_b0_1, enc_W0_2, enc_b0_2,
           enc_W1_0, enc_b1_0, enc_W1_1, enc_b1_1, enc_W1_2, enc_b1_2,
           actor_W0, actor_b0, actor_W1, actor_b1, actor_W2, actor_b2,
           critic_W0, critic_b0, critic_W1, critic_b1, pooled_machine):
    B = x.shape[0]
    f32 = jnp.float32
    off = 2
    feats = x[:, off:off + N * D].reshape(B, N, D)
    off += N * D
    adj = x[:, off:off + N * N].reshape(B, N, N)
    off += N * N
    cand = x[:, off:off + N].astype(jnp.int32).reshape(B, N, 1)
    off += N
    mask = x[:, off:off + N].reshape(B, N, 1)
    act3 = action.astype(jnp.int32).reshape(B, 1, 1)

    def row2(v):
        return v.reshape(1, -1).astype(f32)

    per_sample = lambda bs: pl.BlockSpec(bs, lambda b: (b,) + (0,) * (len(bs) - 1))
    shared = lambda arr: pl.BlockSpec(arr.shape, lambda b: (0,) * arr.ndim)

    weights = [enc_W0_0, row2(enc_b0_0), enc_W0_1, row2(enc_b0_1), enc_W0_2, row2(enc_b0_2),
               enc_W1_0, row2(enc_b1_0), enc_W1_1, row2(enc_b1_1), enc_W1_2, row2(enc_b1_2),
               actor_W0, row2(actor_b0), actor_W1, row2(actor_b1), actor_W2, row2(actor_b2),
               critic_W0, row2(critic_b0), critic_W1, row2(critic_b1), row2(pooled_machine)]

    in_specs = [per_sample((1, N, D)), per_sample((1, N, N)),
                per_sample((1, N, 1)), per_sample((1, N, 1)),
                per_sample((1, 1, 1))] + [shared(w) for w in weights]

    out = pl.pallas_call(
        _fused,
        grid=(B,),
        in_specs=in_specs,
        out_specs=pl.BlockSpec((1, 1, 128), lambda b: (b, 0, 0)),
        out_shape=jax.ShapeDtypeStruct((B, 1, 128), f32),
        compiler_params=pltpu.CompilerParams(
            dimension_semantics=("parallel",),
            vmem_limit_bytes=120 * 1024 * 1024),
    )(feats, adj, cand, mask, act3, *weights)

    return action, out[:, 0, 0], out[:, 0, 1], out[:, 0, 2:3]
